# baseline (speedup 1.0000x reference)
import dataclasses
import functools

import jax
import jax.numpy as jnp
from jax import lax
from jax.experimental import pallas as pl
from jax.experimental.pallas import tpu as pltpu
from jax.experimental.pallas import tpu_sc as plsc

BATCH = 16384
EMB_DIM = 512
L = 16
NC, NS = 2, 16
NW = NC * NS
BPW = BATCH // NW
CHUNK = 16
NCHUNK = BPW // CHUNK
DCHUNK = EMB_DIM // L
NUSER = 100000
AW = 16
ABLK = (3 * NUSER + AW - 1) // AW + 1

_GDN = jax.lax.GatherDimensionNumbers(
    offset_dims=(), collapsed_slice_dims=(0,), start_index_map=(0,))


def _lane_bcast(v, idx16):
    return lax.gather(v, idx16[:, None], dimension_numbers=_GDN,
                      slice_sizes=(1,),
                      mode=lax.GatherScatterMode.PROMISE_IN_BOUNDS)


def _sc_kernel(u_hbm, i_hbm, j_hbm, ue_hbm, al_hbm, ii_hbm, it_hbm, ik_hbm,
               out_hbm,
               idx_u, idx_i, idx_j,
               bufs0, bufs1,
               out_v, sem0, sem1):
    wid = lax.axis_index("s") * NC + lax.axis_index("c")
    base = wid * BPW
    pltpu.sync_copy(u_hbm.at[pl.ds(base, BPW)], idx_u)
    pltpu.sync_copy(i_hbm.at[pl.ds(base, BPW)], idx_i)
    pltpu.sync_copy(j_hbm.at[pl.ds(base, BPW)], idx_j)

    lane = lax.iota(jnp.int32, L)

    def descs(c, bufs, sem):
        off = c * CHUNK
        iu = idx_u.at[pl.ds(off, CHUNK)]
        ii_ = idx_i.at[pl.ds(off, CHUNK)]
        ij = idx_j.at[pl.ds(off, CHUNK)]
        uvec = idx_u[pl.ds(off, CHUNK)]
        u_rows, a0_r, a1_r, a2_r, ii_r, it_r, ik_r, ji_r, jt_r, jk_r = bufs
        cps = [
            pltpu.make_async_copy(ue_hbm.at[iu], u_rows, sem),
            pltpu.make_async_copy(ii_hbm.at[ii_], ii_r, sem),
            pltpu.make_async_copy(it_hbm.at[ii_], it_r, sem),
            pltpu.make_async_copy(ik_hbm.at[ii_], ik_r, sem),
            pltpu.make_async_copy(ii_hbm.at[ij], ji_r, sem),
            pltpu.make_async_copy(it_hbm.at[ij], jt_r, sem),
            pltpu.make_async_copy(ik_hbm.at[ij], jk_r, sem),
        ]
        for k, a_r in enumerate((a0_r, a1_r, a2_r)):
            blk = lax.shift_right_logical(uvec + k * NUSER, 4)
            cps.append(pltpu.make_async_copy(al_hbm.at[blk], a_r, sem))
        return cps

    def issue(c, bufs, sem):
        for d in descs(c, bufs, sem):
            d.start()

    def drain(c, bufs, sem):
        for d in descs(c, bufs, sem):
            d.wait()

    def compute(c, bufs):
        off = c * CHUNK
        u_rows, a0_r, a1_r, a2_r, ii_r, it_r, ik_r, ji_r, jt_r, jk_r = bufs
        uvec = idx_u[pl.ds(off, CHUNK)]
        a0v = plsc.load_gather(a0_r, [lane, uvec & (AW - 1)])
        a1v = plsc.load_gather(a1_r, [lane, (uvec + NUSER) & (AW - 1)])
        a2v = plsc.load_gather(a2_r, [lane, (uvec + 2 * NUSER) & (AW - 1)])
        mx = jnp.maximum(jnp.maximum(a0v, a1v), a2v)
        e0 = jnp.exp(a0v - mx)
        e1 = jnp.exp(a1v - mx)
        e2 = jnp.exp(a2v - mx)
        rs = 1.0 / (e0 + e1 + e2)
        w0v = e0 * rs
        w1v = e1 * rs
        w2v = e2 * rs

        def elem_body(b, res_vec):
            bvec = jnp.full((L,), b, jnp.int32)
            w0 = _lane_bcast(w0v, bvec)
            w1 = _lane_bcast(w1v, bvec)
            w2 = _lane_bcast(w2v, bvec)

            def dim_body(d, acc):
                sl = pl.ds(d * L, L)
                uv = u_rows[b, sl]
                return acc + uv * (w0 * (ii_r[b, sl] - ji_r[b, sl])
                                   + w1 * (it_r[b, sl] - jt_r[b, sl])
                                   + w2 * (ik_r[b, sl] - jk_r[b, sl]))

            acc = lax.fori_loop(0, DCHUNK, dim_body,
                                jnp.zeros((L,), jnp.float32), unroll=4)
            res = jnp.sum(acc)
            return jnp.where(lane == b, res, res_vec)

        res_vec = lax.fori_loop(0, CHUNK, elem_body,
                                jnp.zeros((L,), jnp.float32))
        out_v[pl.ds(off, CHUNK)] = res_vec

    issue(0, bufs0, sem0)

    @pl.loop(0, NCHUNK, step=2)
    def _pair(c):
        issue(c + 1, bufs1, sem1)
        drain(c, bufs0, sem0)
        compute(c, bufs0)

        @pl.when(c + 2 < NCHUNK)
        def _():
            issue(c + 2, bufs0, sem0)

        drain(c + 1, bufs1, sem1)
        compute(c + 1, bufs1)

    pltpu.sync_copy(out_v, out_hbm.at[pl.ds(base, BPW)])


def kernel(u, i, j, user_emb, alpha_emb, item_embed_img, item_embed_txt,
           item_embed_kg):
    aflat = jnp.pad(alpha_emb.T.reshape(-1), (0, ABLK * AW - 3 * NUSER))
    albl = aflat.reshape(ABLK, AW)
    mesh = plsc.VectorSubcoreMesh(core_axis_name="c", subcore_axis_name="s")

    cp = pltpu.CompilerParams(use_tc_tiling_on_sc=False)
    if "needs_layout_passes" in pltpu.CompilerParams.__dataclass_fields__:
        cp = dataclasses.replace(cp, needs_layout_passes=False)

    rowset = [pltpu.VMEM((CHUNK, EMB_DIM), jnp.float32)] + \
             [pltpu.VMEM((CHUNK, AW), jnp.float32)] * 3 + \
             [pltpu.VMEM((CHUNK, EMB_DIM), jnp.float32)] * 6

    run = functools.partial(
        pl.kernel,
        out_type=jax.ShapeDtypeStruct((BATCH,), jnp.float32),
        mesh=mesh,
        compiler_params=cp,
        scratch_types=[
            pltpu.VMEM((BPW,), jnp.int32),
            pltpu.VMEM((BPW,), jnp.int32),
            pltpu.VMEM((BPW,), jnp.int32),
            rowset,
            rowset,
            pltpu.VMEM((BPW,), jnp.float32),
            pltpu.SemaphoreType.DMA,
            pltpu.SemaphoreType.DMA,
        ],
    )(_sc_kernel)
    return run(u.astype(jnp.int32), i.astype(jnp.int32), j.astype(jnp.int32),
               user_emb, albl, item_embed_img, item_embed_txt,
               item_embed_kg)


# speedup vs baseline: 5.5018x; 5.5018x over previous
import dataclasses
import functools

import jax
import jax.numpy as jnp
from jax import lax
from jax.experimental import pallas as pl
from jax.experimental.pallas import tpu as pltpu
from jax.experimental.pallas import tpu_sc as plsc

BATCH = 16384
EMB_DIM = 512
L = 16
NC, NS = 2, 16
NW = NC * NS
BPW = BATCH // NW
CHUNK = 16
NCHUNK = BPW // CHUNK
DCHUNK = EMB_DIM // L
NUSER = 100000
ABLK = (3 * NUSER + 127) // 128 + 1

_GDN = jax.lax.GatherDimensionNumbers(
    offset_dims=(), collapsed_slice_dims=(0,), start_index_map=(0,))


def _lane_bcast(v, idx16):
    return lax.gather(v, idx16[:, None], dimension_numbers=_GDN,
                      slice_sizes=(1,),
                      mode=lax.GatherScatterMode.PROMISE_IN_BOUNDS)


def _sc_kernel(u_hbm, ij_hbm, ab_hbm, ue_hbm, al_hbm, ii_hbm, it_hbm, ik_hbm,
               out_hbm,
               idx_u, idx_ij, idx_ab,
               bufs0, bufs1,
               out_v, sem0, sem1):
    wid = lax.axis_index("s") * NC + lax.axis_index("c")
    base = wid * BPW
    pltpu.sync_copy(u_hbm.at[pl.ds(base, BPW)], idx_u)
    pltpu.sync_copy(ij_hbm.at[pl.ds(2 * base, 2 * BPW)], idx_ij)
    pltpu.sync_copy(ab_hbm.at[pl.ds(3 * base, 3 * BPW)], idx_ab)

    lane = lax.iota(jnp.int32, L)

    def descs(c, bufs, sem):
        off = c * CHUNK
        u_rows, a_r, ii_r, it_r, ik_r = bufs
        return [
            pltpu.make_async_copy(ue_hbm.at[idx_u.at[pl.ds(off, CHUNK)]],
                                  u_rows, sem),
            pltpu.make_async_copy(al_hbm.at[idx_ab.at[pl.ds(3 * off,
                                                            3 * CHUNK)]],
                                  a_r, sem),
            pltpu.make_async_copy(ii_hbm.at[idx_ij.at[pl.ds(2 * off,
                                                            2 * CHUNK)]],
                                  ii_r, sem),
            pltpu.make_async_copy(it_hbm.at[idx_ij.at[pl.ds(2 * off,
                                                            2 * CHUNK)]],
                                  it_r, sem),
            pltpu.make_async_copy(ik_hbm.at[idx_ij.at[pl.ds(2 * off,
                                                            2 * CHUNK)]],
                                  ik_r, sem),
        ]

    def issue(c, bufs, sem):
        for d in descs(c, bufs, sem):
            d.start()

    def drain(c, bufs, sem):
        for d in descs(c, bufs, sem):
            d.wait()

    def compute(c, bufs):
        off = c * CHUNK
        u_rows, a_r, ii_r, it_r, ik_r = bufs
        uvec = idx_u[pl.ds(off, CHUNK)]
        a0v = plsc.load_gather(a_r, [lane, uvec & 127])
        a1v = plsc.load_gather(a_r, [lane + CHUNK, (uvec + NUSER) & 127])
        a2v = plsc.load_gather(a_r, [lane + 2 * CHUNK,
                                     (uvec + 2 * NUSER) & 127])
        mx = jnp.maximum(jnp.maximum(a0v, a1v), a2v)
        e0 = jnp.exp(a0v - mx)
        e1 = jnp.exp(a1v - mx)
        e2 = jnp.exp(a2v - mx)
        rs = 1.0 / (e0 + e1 + e2)
        w0v = e0 * rs
        w1v = e1 * rs
        w2v = e2 * rs

        def elem_body(b, res_vec):
            bvec = jnp.full((L,), b, jnp.int32)
            w0 = _lane_bcast(w0v, bvec)
            w1 = _lane_bcast(w1v, bvec)
            w2 = _lane_bcast(w2v, bvec)
            bj = b + CHUNK

            def dim_body(d, acc):
                sl = pl.ds(d * L, L)
                uv = u_rows[b, sl]
                return acc + uv * (w0 * (ii_r[b, sl] - ii_r[bj, sl])
                                   + w1 * (it_r[b, sl] - it_r[bj, sl])
                                   + w2 * (ik_r[b, sl] - ik_r[bj, sl]))

            acc = lax.fori_loop(0, DCHUNK, dim_body,
                                jnp.zeros((L,), jnp.float32), unroll=4)
            res = jnp.sum(acc)
            return jnp.where(lane == b, res, res_vec)

        res_vec = lax.fori_loop(0, CHUNK, elem_body,
                                jnp.zeros((L,), jnp.float32))
        out_v[pl.ds(off, CHUNK)] = res_vec

    issue(0, bufs0, sem0)

    @pl.loop(0, NCHUNK, step=2)
    def _pair(c):
        issue(c + 1, bufs1, sem1)
        drain(c, bufs0, sem0)
        compute(c, bufs0)

        @pl.when(c + 2 < NCHUNK)
        def _():
            issue(c + 2, bufs0, sem0)

        drain(c + 1, bufs1, sem1)
        compute(c + 1, bufs1)

    pltpu.sync_copy(out_v, out_hbm.at[pl.ds(base, BPW)])


def kernel(u, i, j, user_emb, alpha_emb, item_embed_img, item_embed_txt,
           item_embed_kg):
    u = u.astype(jnp.int32)
    i = i.astype(jnp.int32)
    j = j.astype(jnp.int32)
    aflat = jnp.pad(alpha_emb.T.reshape(-1), (0, ABLK * 128 - 3 * NUSER))
    albl = aflat.reshape(ABLK, 128)
    iv = i.reshape(-1, CHUNK)
    jv = j.reshape(-1, CHUNK)
    ij = jnp.stack([iv, jv], axis=1).reshape(-1)
    au = u.reshape(-1, CHUNK)
    ab = jnp.stack([(au + k * NUSER) >> 7 for k in range(3)],
                   axis=1).reshape(-1)

    mesh = plsc.VectorSubcoreMesh(core_axis_name="c", subcore_axis_name="s")

    cp = pltpu.CompilerParams()
    if "needs_layout_passes" in pltpu.CompilerParams.__dataclass_fields__:
        cp = dataclasses.replace(cp, needs_layout_passes=False)

    rowset = [pltpu.VMEM((CHUNK, EMB_DIM), jnp.float32),
              pltpu.VMEM((3 * CHUNK, 128), jnp.float32)] + \
             [pltpu.VMEM((2 * CHUNK, EMB_DIM), jnp.float32)] * 3

    run = functools.partial(
        pl.kernel,
        out_type=jax.ShapeDtypeStruct((BATCH,), jnp.float32),
        mesh=mesh,
        compiler_params=cp,
        scratch_types=[
            pltpu.VMEM((BPW,), jnp.int32),
            pltpu.VMEM((2 * BPW,), jnp.int32),
            pltpu.VMEM((3 * BPW,), jnp.int32),
            rowset,
            rowset,
            pltpu.VMEM((BPW,), jnp.float32),
            pltpu.SemaphoreType.DMA,
            pltpu.SemaphoreType.DMA,
        ],
    )(_sc_kernel)
    return run(u, ij, ab, user_emb, albl, item_embed_img, item_embed_txt,
               item_embed_kg)


# speedup vs baseline: 5.7800x; 1.0506x over previous
import dataclasses
import functools

import jax
import jax.numpy as jnp
from jax import lax
from jax.experimental import pallas as pl
from jax.experimental.pallas import tpu as pltpu
from jax.experimental.pallas import tpu_sc as plsc

BATCH = 16384
EMB_DIM = 512
L = 16
NC, NS = 2, 16
NW = NC * NS
BPW = BATCH // NW
CHUNK = 16
NCHUNK = BPW // CHUNK
DCHUNK = EMB_DIM // L
NUSER = 100000
ABLK = (3 * NUSER + 127) // 128 + 1

_GDN = jax.lax.GatherDimensionNumbers(
    offset_dims=(), collapsed_slice_dims=(0,), start_index_map=(0,))


def _lane_bcast(v, idx16):
    return lax.gather(v, idx16[:, None], dimension_numbers=_GDN,
                      slice_sizes=(1,),
                      mode=lax.GatherScatterMode.PROMISE_IN_BOUNDS)


def _sc_kernel(u_hbm, i_hbm, j_hbm, ue_hbm, al_hbm, ii_hbm, it_hbm, ik_hbm,
               out_hbm,
               idx_u, idx_i, idx_j,
               bufs0, bufs1,
               out_v, sem0, sem1):
    wid = lax.axis_index("s") * NC + lax.axis_index("c")
    base = wid * BPW
    pltpu.sync_copy(u_hbm.at[pl.ds(base, BPW)], idx_u)
    pltpu.sync_copy(i_hbm.at[pl.ds(base, BPW)], idx_i)
    pltpu.sync_copy(j_hbm.at[pl.ds(base, BPW)], idx_j)

    lane = lax.iota(jnp.int32, L)

    def descs(c, bufs, sem):
        off = c * CHUNK
        iu = idx_u.at[pl.ds(off, CHUNK)]
        ii_ = idx_i.at[pl.ds(off, CHUNK)]
        ij = idx_j.at[pl.ds(off, CHUNK)]
        uvec = idx_u[pl.ds(off, CHUNK)]
        u_rows, a0_r, a1_r, a2_r, ii_r, it_r, ik_r, ji_r, jt_r, jk_r = bufs
        cps = [
            pltpu.make_async_copy(ue_hbm.at[iu], u_rows, sem),
            pltpu.make_async_copy(ii_hbm.at[ii_], ii_r, sem),
            pltpu.make_async_copy(it_hbm.at[ii_], it_r, sem),
            pltpu.make_async_copy(ik_hbm.at[ii_], ik_r, sem),
            pltpu.make_async_copy(ii_hbm.at[ij], ji_r, sem),
            pltpu.make_async_copy(it_hbm.at[ij], jt_r, sem),
            pltpu.make_async_copy(ik_hbm.at[ij], jk_r, sem),
        ]
        for k, a_r in enumerate((a0_r, a1_r, a2_r)):
            blk = lax.shift_right_logical(uvec + k * NUSER, 7)
            cps.append(pltpu.make_async_copy(al_hbm.at[blk], a_r, sem))
        return cps

    def issue(c, bufs, sem):
        for d in descs(c, bufs, sem):
            d.start()

    def drain(c, bufs, sem):
        for d in descs(c, bufs, sem):
            d.wait()

    def compute(c, bufs):
        off = c * CHUNK
        u_rows, a0_r, a1_r, a2_r, ii_r, it_r, ik_r, ji_r, jt_r, jk_r = bufs
        uvec = idx_u[pl.ds(off, CHUNK)]
        a0v = plsc.load_gather(a0_r, [lane, uvec & 127])
        a1v = plsc.load_gather(a1_r, [lane, (uvec + NUSER) & 127])
        a2v = plsc.load_gather(a2_r, [lane, (uvec + 2 * NUSER) & 127])
        mx = jnp.maximum(jnp.maximum(a0v, a1v), a2v)
        e0 = jnp.exp(a0v - mx)
        e1 = jnp.exp(a1v - mx)
        e2 = jnp.exp(a2v - mx)
        rs = 1.0 / (e0 + e1 + e2)
        w0v = e0 * rs
        w1v = e1 * rs
        w2v = e2 * rs

        def elem_body(b, res_vec):
            bvec = jnp.full((L,), b, jnp.int32)
            w0 = _lane_bcast(w0v, bvec)
            w1 = _lane_bcast(w1v, bvec)
            w2 = _lane_bcast(w2v, bvec)

            def dim_body(d, acc):
                sl = pl.ds(d * L, L)
                uv = u_rows[b, sl]
                return acc + uv * (w0 * (ii_r[b, sl] - ji_r[b, sl])
                                   + w1 * (it_r[b, sl] - jt_r[b, sl])
                                   + w2 * (ik_r[b, sl] - jk_r[b, sl]))

            acc = lax.fori_loop(0, DCHUNK, dim_body,
                                jnp.zeros((L,), jnp.float32), unroll=4)
            res = jnp.sum(acc)
            return jnp.where(lane == b, res, res_vec)

        res_vec = lax.fori_loop(0, CHUNK, elem_body,
                                jnp.zeros((L,), jnp.float32))
        out_v[pl.ds(off, CHUNK)] = res_vec

    issue(0, bufs0, sem0)

    @pl.loop(0, NCHUNK, step=2)
    def _pair(c):
        issue(c + 1, bufs1, sem1)
        drain(c, bufs0, sem0)
        compute(c, bufs0)

        @pl.when(c + 2 < NCHUNK)
        def _():
            issue(c + 2, bufs0, sem0)

        drain(c + 1, bufs1, sem1)
        compute(c + 1, bufs1)

    pltpu.sync_copy(out_v, out_hbm.at[pl.ds(base, BPW)])


def kernel(u, i, j, user_emb, alpha_emb, item_embed_img, item_embed_txt,
           item_embed_kg):
    aflat = jnp.pad(alpha_emb.T.reshape(-1), (0, ABLK * 128 - 3 * NUSER))
    albl = aflat.reshape(ABLK, 128)
    mesh = plsc.VectorSubcoreMesh(core_axis_name="c", subcore_axis_name="s")

    cp = pltpu.CompilerParams()
    if "needs_layout_passes" in pltpu.CompilerParams.__dataclass_fields__:
        cp = dataclasses.replace(cp, needs_layout_passes=False)

    rowset = [pltpu.VMEM((CHUNK, EMB_DIM), jnp.float32)] + \
             [pltpu.VMEM((CHUNK, 128), jnp.float32)] * 3 + \
             [pltpu.VMEM((CHUNK, EMB_DIM), jnp.float32)] * 6

    run = functools.partial(
        pl.kernel,
        out_type=jax.ShapeDtypeStruct((BATCH,), jnp.float32),
        mesh=mesh,
        compiler_params=cp,
        scratch_types=[
            pltpu.VMEM((BPW,), jnp.int32),
            pltpu.VMEM((BPW,), jnp.int32),
            pltpu.VMEM((BPW,), jnp.int32),
            rowset,
            rowset,
            pltpu.VMEM((BPW,), jnp.float32),
            pltpu.SemaphoreType.DMA,
            pltpu.SemaphoreType.DMA,
        ],
    )(_sc_kernel)
    return run(u.astype(jnp.int32), i.astype(jnp.int32), j.astype(jnp.int32),
               user_emb, albl, item_embed_img, item_embed_txt,
               item_embed_kg)
